# baseline (speedup 1.0000x reference)
import functools

import jax
import jax.numpy as jnp
from jax import lax
from jax.experimental import pallas as pl
from jax.experimental.pallas import tpu as pltpu
from jax.experimental.pallas import tpu_sc as plsc

N = 10000
E = 160000
NSUB = 16
BLK = 128
NBLK = 80
E_PAD = NSUB * NBLK * BLK
NACC = 10240
STRIPE = NACC // NSUB
CW = 128
WIN = 40
NWIN = NBLK // WIN
NBUF = 2
ZROWS = 32

NC1 = 6
NC2 = 4

MM_BN = 1000
MM_GRID = N // MM_BN

f32 = jnp.float32


def _mesh():
    return plsc.VectorSubcoreMesh(core_axis_name="c", subcore_axis_name="s")


def _segsum_scratch():
    return [
        pltpu.VMEM((WIN, BLK), jnp.int32),
        pltpu.VMEM((WIN, BLK), jnp.int32),
        pltpu.VMEM((NBUF, BLK, CW), f32),
        pltpu.VMEM((ZROWS, CW), f32),
        pltpu.VMEM_SHARED((NACC, CW), f32),
        pltpu.SemaphoreType.DMA,
        pltpu.SemaphoreType.DMA,
    ]


def _fill_zeros(zbuf):
    zv = jnp.zeros((16,), f32)

    def row(i, carry):
        for j in range(CW // 16):
            zbuf[i, pl.ds(j * 16, 16)] = zv
        return carry

    lax.fori_loop(0, ZROWS, row, 0)


def _zero_stripe(zbuf, acc, t):
    for s in range(STRIPE // ZROWS):
        pltpu.sync_copy(zbuf, acc.at[pl.ds(t * STRIPE + s * ZROWS, ZROWS)])


def _edge_window(z_ref, sidx, didx, rows, acc, sems):
    ngrp = WIN // NBUF

    def gather(j, b):
        return pltpu.async_copy(z_ref.at[sidx.at[j]], rows.at[b], sems[b])

    def wait(b):
        pltpu.make_async_copy(z_ref.at[sidx.at[0]], rows.at[b], sems[b]).wait()

    for b in range(NBUF):
        gather(b, b)

    def step(m, carry):
        j = NBUF * m
        for b in range(NBUF):
            wait(b)
            pltpu.sync_copy(rows.at[b], acc.at[didx.at[j + b]], add=True)

            @pl.when(m < ngrp - 1)
            def _(b=b):
                gather(j + NBUF + b, b)
        return carry

    lax.fori_loop(0, ngrp, step, 0)


def _edge_loop(z_ref, src_ref, dst_ref, t, sidx, didx, rows, acc, sems, windows):
    for w in windows:
        pltpu.sync_copy(src_ref.at[t, pl.ds(w * WIN, WIN)], sidx)
        pltpu.sync_copy(dst_ref.at[t, pl.ds(w * WIN, WIN)], didx)
        _edge_window(z_ref, sidx, didx, rows, acc, sems)


def _make_segsum_pair():
    out_type = (jax.ShapeDtypeStruct((NACC, CW), f32),
                jax.ShapeDtypeStruct((NACC, CW), f32))

    def body(z0, z1, src_ref, dst_ref, out0, out1,
             sidx, didx, rows, zbuf, acc, semA, semB):
        core = lax.axis_index("c")
        t = lax.axis_index("s")
        _fill_zeros(zbuf)
        _zero_stripe(zbuf, acc, t)
        plsc.subcore_barrier()
        for c, (z_ref, out_ref) in enumerate(((z0, out0), (z1, out1))):
            @pl.when(core == c)
            def _(z_ref=z_ref, out_ref=out_ref):
                _edge_loop(z_ref, src_ref, dst_ref, t, sidx, didx, rows,
                           acc, (semA, semB), range(NWIN))
                plsc.subcore_barrier()
                pltpu.sync_copy(acc.at[pl.ds(t * STRIPE, STRIPE)],
                                out_ref.at[pl.ds(t * STRIPE, STRIPE)])

    return pl.kernel(body, out_type=out_type, mesh=_mesh(),
                     scratch_types=_segsum_scratch())


def _make_segsum_edges():
    out_type = (jax.ShapeDtypeStruct((NACC, CW), f32),
                jax.ShapeDtypeStruct((NACC, CW), f32))

    def body(z_ref, src_ref, dst_ref, out0, out1,
             sidx, didx, rows, zbuf, acc, semA, semB):
        core = lax.axis_index("c")
        t = lax.axis_index("s")
        _fill_zeros(zbuf)
        _zero_stripe(zbuf, acc, t)
        plsc.subcore_barrier()
        for k, out_ref in enumerate((out0, out1)):
            @pl.when(core == k)
            def _(k=k, out_ref=out_ref):
                _edge_loop(z_ref, src_ref, dst_ref, t, sidx, didx, rows,
                           acc, (semA, semB),
                           range(k * (NWIN // 2), (k + 1) * (NWIN // 2)))
                plsc.subcore_barrier()
                pltpu.sync_copy(acc.at[pl.ds(t * STRIPE, STRIPE)],
                                out_ref.at[pl.ds(t * STRIPE, STRIPE)])

    return pl.kernel(body, out_type=out_type, mesh=_mesh(),
                     scratch_types=_segsum_scratch())


def _mm1_pair_body(x_ref, w_ref, o0, o1):
    z = jnp.dot(x_ref[...], w_ref[...], preferred_element_type=f32)
    o0[...] = z[:, :CW]
    o1[...] = z[:, CW:]


def _pair_partial(a0, a1, b_ref, w_ref):
    x0 = jnp.maximum(a0[...] + b_ref[0, :CW], 0.0)
    x1 = jnp.maximum(a1[...] + b_ref[0, CW:], 0.0)
    return (jnp.dot(x0, w_ref[:CW, :], preferred_element_type=f32)
            + jnp.dot(x1, w_ref[CW:, :], preferred_element_type=f32))


def _mm2_body(init, *refs):
    if init:
        a0, a1, b_ref, w_ref = refs[:4]
        out_refs = refs[4:]
        part = _pair_partial(a0, a1, b_ref, w_ref)
        for c in range(NC2):
            out_refs[c][...] = part[:, c * CW:(c + 1) * CW]
    else:
        zin = refs[:NC2]
        a0, a1, b_ref, w_ref = refs[NC2:NC2 + 4]
        out_refs = refs[NC2 + 4:]
        part = _pair_partial(a0, a1, b_ref, w_ref)
        for c in range(NC2):
            out_refs[c][...] = zin[c][...] + part[:, c * CW:(c + 1) * CW]


def _mm3_body(init, *refs):
    if init:
        a0, a1, b_ref, w_ref, wi_ref = refs[:5]
        out_ref = refs[5]
        prev = 0.0
    else:
        zin, a0, a1, b_ref, w_ref, wi_ref, out_ref = refs
        prev = zin[...]
    part = _pair_partial(a0, a1, b_ref, w_ref)
    out_ref[...] = prev + jnp.dot(part, wi_ref[...], preferred_element_type=f32)


def _final_body(p0, p1, b3_ref, wi_ref, bi_ref, out_ref):
    bv = jnp.dot(b3_ref[...], wi_ref[...], preferred_element_type=f32) + bi_ref[...]
    h = p0[:, :64] + p1[:, :64] + bv
    out_ref[...] = jnp.maximum(h, 0.0)


def _row_spec(shape):
    return pl.BlockSpec((MM_BN,) + shape[1:], lambda i: (i,) + (0,) * (len(shape) - 1))


def _full_spec(shape):
    return pl.BlockSpec(shape, lambda i: (0,) * len(shape))


def kernel(features, edge_index, W1, b1, W2, b2, W3, b3, Wi, bi):
    W1p = jnp.pad(W1, ((0, 0), (0, NC1 * CW - 700)))
    b1p = jnp.pad(b1, (0, NC1 * CW - 700)).reshape(1, NC1 * CW)
    W2p = jnp.pad(W2, ((0, NC1 * CW - 700), (0, NC2 * CW - 400)))
    b2p = jnp.pad(b2, (0, NC2 * CW - 400)).reshape(1, NC2 * CW)
    W3p = jnp.pad(W3, ((0, NC2 * CW - 400), (0, 0)))
    Wip = jnp.pad(Wi, ((0, 0), (0, CW - 64)))
    b3r = b3.reshape(1, 100)
    bir = bi.reshape(1, 64)

    src = edge_index[0]
    dst = edge_index[1]
    pad = E_PAD - E
    srcp = jnp.concatenate([src, jnp.zeros((pad,), jnp.int32)]).reshape(NSUB, NBLK, BLK)
    dump = N + (jnp.arange(pad, dtype=jnp.int32) % (NACC - N))
    dstp = jnp.concatenate([dst, dump]).reshape(NSUB, NBLK, BLK)

    segsum_pair = _make_segsum_pair()
    segsum_edges = _make_segsum_edges()

    mm1 = pl.pallas_call(
        _mm1_pair_body,
        grid=(MM_GRID,),
        in_specs=[_row_spec((N, 1433)), _full_spec((1433, 2 * CW))],
        out_specs=[_row_spec((N, CW))] * 2,
        out_shape=[jax.ShapeDtypeStruct((N, CW), f32)] * 2,
    )
    a1 = []
    for p in range(NC1 // 2):
        zp = mm1(features, W1p[:, 2 * p * CW:2 * (p + 1) * CW])
        a1.append(segsum_pair(zp[0], zp[1], srcp, dstp))

    mm2_init = pl.pallas_call(
        functools.partial(_mm2_body, True),
        grid=(MM_GRID,),
        in_specs=[_row_spec((NACC, CW))] * 2 + [
            _full_spec((1, 2 * CW)), _full_spec((2 * CW, NC2 * CW))],
        out_specs=[_row_spec((N, CW))] * NC2,
        out_shape=[jax.ShapeDtypeStruct((N, CW), f32)] * NC2,
    )
    mm2_acc = pl.pallas_call(
        functools.partial(_mm2_body, False),
        grid=(MM_GRID,),
        in_specs=[_row_spec((N, CW))] * NC2 + [_row_spec((NACC, CW))] * 2 + [
            _full_spec((1, 2 * CW)), _full_spec((2 * CW, NC2 * CW))],
        out_specs=[_row_spec((N, CW))] * NC2,
        out_shape=[jax.ShapeDtypeStruct((N, CW), f32)] * NC2,
        input_output_aliases={c: c for c in range(NC2)},
    )
    z2 = None
    for p in range(NC1 // 2):
        bp = b1p[:, 2 * p * CW:2 * (p + 1) * CW]
        wp = W2p[2 * p * CW:2 * (p + 1) * CW, :]
        if z2 is None:
            z2 = mm2_init(a1[p][0], a1[p][1], bp, wp)
        else:
            z2 = mm2_acc(*z2, a1[p][0], a1[p][1], bp, wp)

    a2 = []
    for p in range(NC2 // 2):
        a2.append(segsum_pair(z2[2 * p], z2[2 * p + 1], srcp, dstp))

    mm3_init = pl.pallas_call(
        functools.partial(_mm3_body, True),
        grid=(MM_GRID,),
        in_specs=[_row_spec((NACC, CW))] * 2 + [
            _full_spec((1, 2 * CW)), _full_spec((2 * CW, 100)),
            _full_spec((100, CW))],
        out_specs=_row_spec((N, CW)),
        out_shape=jax.ShapeDtypeStruct((N, CW), f32),
    )
    mm3_acc = pl.pallas_call(
        functools.partial(_mm3_body, False),
        grid=(MM_GRID,),
        in_specs=[_row_spec((N, CW))] + [_row_spec((NACC, CW))] * 2 + [
            _full_spec((1, 2 * CW)), _full_spec((2 * CW, 100)),
            _full_spec((100, CW))],
        out_specs=_row_spec((N, CW)),
        out_shape=jax.ShapeDtypeStruct((N, CW), f32),
        input_output_aliases={0: 0},
    )
    z3 = None
    for p in range(NC2 // 2):
        bp = b2p[:, 2 * p * CW:2 * (p + 1) * CW]
        wp = W3p[2 * p * CW:2 * (p + 1) * CW, :]
        if z3 is None:
            z3 = mm3_init(a2[p][0], a2[p][1], bp, wp, Wip)
        else:
            z3 = mm3_acc(z3, a2[p][0], a2[p][1], bp, wp, Wip)

    a3 = segsum_edges(z3, srcp, dstp)

    fin = pl.pallas_call(
        _final_body,
        grid=(MM_GRID,),
        in_specs=[_row_spec((NACC, CW))] * 2 + [
            _full_spec((1, 100)), _full_spec((100, 64)), _full_spec((1, 64))],
        out_specs=_row_spec((N, 64)),
        out_shape=jax.ShapeDtypeStruct((N, 64), f32),
    )
    return fin(*a3, b3r, Wi, bir)


# speedup vs baseline: 1.0596x; 1.0596x over previous
import functools

import jax
import jax.numpy as jnp
from jax import lax
from jax.experimental import pallas as pl
from jax.experimental.pallas import tpu as pltpu
from jax.experimental.pallas import tpu_sc as plsc

N = 10000
E = 160000
NSUB = 16
BLK = 128
NBLK = 80
E_PAD = NSUB * NBLK * BLK
NACC = 10240
STRIPE = NACC // NSUB
CW = 128
WIN = 40
NWIN = NBLK // WIN

NC1 = 6
NC2 = 4

MM_BN = 1000
MM_GRID = N // MM_BN


def _mesh():
    return plsc.VectorSubcoreMesh(core_axis_name="c", subcore_axis_name="s")


def _segsum_scratch():
    return [
        pltpu.VMEM((WIN, BLK), jnp.int32),
        pltpu.VMEM((WIN, BLK), jnp.int32),
        pltpu.VMEM((2, BLK, CW), jnp.float32),
        pltpu.VMEM((32, CW), jnp.float32),
        pltpu.VMEM_SHARED((NACC, CW), jnp.float32),
        pltpu.SemaphoreType.DMA,
        pltpu.SemaphoreType.DMA,
    ]


ZROWS = 32


def _fill_zeros(zbuf):
    zv = jnp.zeros((16,), jnp.float32)

    def row(i, carry):
        for j in range(CW // 16):
            zbuf[i, pl.ds(j * 16, 16)] = zv
        return carry

    lax.fori_loop(0, ZROWS, row, 0)


def _zero_stripe(zbuf, acc, t):
    for s in range(STRIPE // ZROWS):
        pltpu.sync_copy(zbuf, acc.at[pl.ds(t * STRIPE + s * ZROWS, ZROWS)])


def _edge_window(z_ref, sidx, didx, rows, acc, sems):
    npairs = WIN // 2

    def gather(j, b):
        return pltpu.async_copy(z_ref.at[sidx.at[j]], rows.at[b], sems[b])

    def wait(b):
        pltpu.make_async_copy(z_ref.at[sidx.at[0]], rows.at[b], sems[b]).wait()

    gather(0, 0)
    gather(1, 1)

    def step(m, carry):
        j = 2 * m
        for b in range(2):
            wait(b)
            pltpu.sync_copy(rows.at[b], acc.at[didx.at[j + b]], add=True)

            @pl.when(m < npairs - 1)
            def _(b=b):
                gather(j + 2 + b, b)
        return carry

    lax.fori_loop(0, npairs, step, 0)


def _edge_loop(z_ref, src_ref, dst_ref, t, sidx, didx, rows, acc, sems, windows):
    for w in windows:
        pltpu.sync_copy(src_ref.at[t, pl.ds(w * WIN, WIN)], sidx)
        pltpu.sync_copy(dst_ref.at[t, pl.ds(w * WIN, WIN)], didx)
        _edge_window(z_ref, sidx, didx, rows, acc, sems)


def _make_segsum_cols(nchunks):
    out_type = tuple(jax.ShapeDtypeStruct((NACC, CW), jnp.float32)
                     for _ in range(nchunks))

    def body(*refs):
        z_refs = refs[:nchunks]
        src_ref, dst_ref = refs[nchunks:nchunks + 2]
        out_refs = refs[nchunks + 2:2 * nchunks + 2]
        sidx, didx, rows, zbuf, acc, semA, semB = refs[2 * nchunks + 2:]
        core = lax.axis_index("c")
        t = lax.axis_index("s")
        _fill_zeros(zbuf)

        for c in range(nchunks):
            @pl.when(core == (c % 2))
            def _(c=c):
                _zero_stripe(zbuf, acc, t)
                plsc.subcore_barrier()
                _edge_loop(z_refs[c], src_ref, dst_ref, t, sidx, didx, rows,
                           acc, (semA, semB), range(NWIN))
                plsc.subcore_barrier()
                pltpu.sync_copy(acc.at[pl.ds(t * STRIPE, STRIPE)],
                                out_refs[c].at[pl.ds(t * STRIPE, STRIPE)])
                plsc.subcore_barrier()

    return pl.kernel(body, out_type=out_type, mesh=_mesh(),
                     scratch_types=_segsum_scratch())


def _make_segsum_edges():
    out_type = (jax.ShapeDtypeStruct((NACC, CW), jnp.float32),
                jax.ShapeDtypeStruct((NACC, CW), jnp.float32))

    def body(z_ref, src_ref, dst_ref, out0, out1,
             sidx, didx, rows, zbuf, acc, semA, semB):
        core = lax.axis_index("c")
        t = lax.axis_index("s")
        _fill_zeros(zbuf)
        _zero_stripe(zbuf, acc, t)
        plsc.subcore_barrier()
        for k, out_ref in enumerate((out0, out1)):
            @pl.when(core == k)
            def _(k=k, out_ref=out_ref):
                _edge_loop(z_ref, src_ref, dst_ref, t, sidx, didx, rows,
                           acc, (semA, semB), [k])
                plsc.subcore_barrier()
                pltpu.sync_copy(acc.at[pl.ds(t * STRIPE, STRIPE)],
                                out_ref.at[pl.ds(t * STRIPE, STRIPE)])

    return pl.kernel(body, out_type=out_type, mesh=_mesh(),
                     scratch_types=_segsum_scratch())


def _mm1_body(x_ref, w_ref, *out_refs):
    z = jnp.dot(x_ref[...], w_ref[...], preferred_element_type=jnp.float32)
    for c in range(NC1):
        out_refs[c][...] = z[:, c * CW:(c + 1) * CW]


def _mm2_body(*refs):
    a_refs = refs[:NC1]
    b_ref, w_ref = refs[NC1:NC1 + 2]
    out_refs = refs[NC1 + 2:]
    acc = None
    for c in range(NC1):
        xc = jnp.maximum(a_refs[c][...] + b_ref[0, c * CW:(c + 1) * CW], 0.0)
        p = jnp.dot(xc, w_ref[c * CW:(c + 1) * CW, :],
                    preferred_element_type=jnp.float32)
        acc = p if acc is None else acc + p
    for c in range(NC2):
        out_refs[c][...] = acc[:, c * CW:(c + 1) * CW]


def _mm3_body(a0, a1, a2, a3, b_ref, w3_ref, wi_ref, out_ref):
    a_refs = (a0, a1, a2, a3)
    acc = None
    for c in range(NC2):
        xc = jnp.maximum(a_refs[c][...] + b_ref[0, c * CW:(c + 1) * CW], 0.0)
        p = jnp.dot(xc, w3_ref[c * CW:(c + 1) * CW, :],
                    preferred_element_type=jnp.float32)
        acc = p if acc is None else acc + p
    out_ref[...] = jnp.dot(acc, wi_ref[...], preferred_element_type=jnp.float32)


def _final_body(p0, p1, b3_ref, wi_ref, bi_ref, out_ref):
    bv = jnp.dot(b3_ref[...], wi_ref[...],
                 preferred_element_type=jnp.float32) + bi_ref[...]
    h = p0[:, :64] + p1[:, :64] + bv
    out_ref[...] = jnp.maximum(h, 0.0)


def _row_spec(shape):
    return pl.BlockSpec((MM_BN,) + shape[1:], lambda i: (i,) + (0,) * (len(shape) - 1))


def _full_spec(shape):
    return pl.BlockSpec(shape, lambda i: (0,) * len(shape))


def kernel(features, edge_index, W1, b1, W2, b2, W3, b3, Wi, bi):
    f32 = jnp.float32
    W1p = jnp.pad(W1, ((0, 0), (0, NC1 * CW - 700)))
    b1p = jnp.pad(b1, (0, NC1 * CW - 700)).reshape(1, NC1 * CW)
    W2p = jnp.pad(W2, ((0, NC1 * CW - 700), (0, NC2 * CW - 400)))
    b2p = jnp.pad(b2, (0, NC2 * CW - 400)).reshape(1, NC2 * CW)
    W3p = jnp.pad(W3, ((0, NC2 * CW - 400), (0, 0)))
    Wip = jnp.pad(Wi, ((0, 0), (0, CW - 64)))
    b3r = b3.reshape(1, 100)
    bir = bi.reshape(1, 64)

    src = edge_index[0]
    dst = edge_index[1]
    pad = E_PAD - E
    srcp = jnp.concatenate([src, jnp.zeros((pad,), jnp.int32)]).reshape(NSUB, NBLK, BLK)
    dump = N + (jnp.arange(pad, dtype=jnp.int32) % (NACC - N))
    dstp = jnp.concatenate([dst, dump]).reshape(NSUB, NBLK, BLK)
    mm1 = pl.pallas_call(
        _mm1_body,
        grid=(MM_GRID,),
        in_specs=[_row_spec((N, 1433)), _full_spec((1433, NC1 * CW))],
        out_specs=[_row_spec((N, CW))] * NC1,
        out_shape=[jax.ShapeDtypeStruct((N, CW), f32)] * NC1,
    )
    z1 = mm1(features, W1p)

    a1 = _make_segsum_cols(NC1)(*z1, srcp, dstp)

    mm2 = pl.pallas_call(
        _mm2_body,
        grid=(MM_GRID,),
        in_specs=[_row_spec((NACC, CW))] * NC1 + [
            _full_spec((1, NC1 * CW)), _full_spec((NC1 * CW, NC2 * CW))],
        out_specs=[_row_spec((N, CW))] * NC2,
        out_shape=[jax.ShapeDtypeStruct((N, CW), f32)] * NC2,
    )
    z2 = mm2(*a1, b1p, W2p)

    a2 = _make_segsum_cols(NC2)(*z2, srcp, dstp)

    mm3 = pl.pallas_call(
        _mm3_body,
        grid=(MM_GRID,),
        in_specs=[_row_spec((NACC, CW))] * NC2 + [
            _full_spec((1, NC2 * CW)), _full_spec((NC2 * CW, 100)),
            _full_spec((100, CW))],
        out_specs=_row_spec((N, CW)),
        out_shape=jax.ShapeDtypeStruct((N, CW), f32),
    )
    z3 = mm3(*a2, b2p, W3p, Wip)

    a3 = _make_segsum_edges()(z3, srcp, dstp)

    fin = pl.pallas_call(
        _final_body,
        grid=(MM_GRID,),
        in_specs=[_row_spec((NACC, CW))] * 2 + [
            _full_spec((1, 100)), _full_spec((100, 64)), _full_spec((1, 64))],
        out_specs=_row_spec((N, 64)),
        out_shape=jax.ShapeDtypeStruct((N, 64), f32),
    )
    return fin(*a3, b3r, Wi, bir)


# speedup vs baseline: 1.0957x; 1.0340x over previous
import functools

import jax
import jax.numpy as jnp
from jax import lax
from jax.experimental import pallas as pl
from jax.experimental.pallas import tpu as pltpu
from jax.experimental.pallas import tpu_sc as plsc

N = 10000
E = 160000
NSUB = 16
BLK = 128
NBLK = 80
E_PAD = NSUB * NBLK * BLK
NACC = 10112
STRIPE = NACC // NSUB
CW = 128
WIN = 40
NWIN = NBLK // WIN

NC1 = 6
NC2 = 4

MM_BN = 1000
MM_GRID = N // MM_BN


def _mesh():
    return plsc.VectorSubcoreMesh(core_axis_name="c", subcore_axis_name="s")


def _segsum_scratch():
    return [
        pltpu.VMEM((WIN, BLK), jnp.int32),
        pltpu.VMEM((WIN, BLK), jnp.int32),
        pltpu.VMEM((2, BLK, CW), jnp.float32),
        pltpu.VMEM_SHARED((NACC, CW), jnp.float32),
        pltpu.SemaphoreType.DMA,
        pltpu.SemaphoreType.DMA,
    ]


def _edge_window(z_ref, sidx, didx, rows, acc, sems):
    npairs = WIN // 2

    def gather(j, b):
        return pltpu.async_copy(z_ref.at[sidx.at[j]], rows.at[b], sems[b])

    def wait(b):
        pltpu.make_async_copy(z_ref.at[sidx.at[0]], rows.at[b], sems[b]).wait()

    gather(0, 0)
    gather(1, 1)

    def step(m, carry):
        j = 2 * m
        for b in range(2):
            wait(b)
            pltpu.sync_copy(rows.at[b], acc.at[didx.at[j + b]], add=True)

            @pl.when(m < npairs - 1)
            def _(b=b):
                gather(j + 2 + b, b)
        return carry

    lax.fori_loop(0, npairs, step, 0)


def _edge_loop(z_ref, src_ref, dst_ref, t, sidx, didx, rows, acc, sems, windows):
    for w in windows:
        pltpu.sync_copy(src_ref.at[t, pl.ds(w * WIN, WIN)], sidx)
        pltpu.sync_copy(dst_ref.at[t, pl.ds(w * WIN, WIN)], didx)
        _edge_window(z_ref, sidx, didx, rows, acc, sems)


def _make_segsum_cols(nchunks):
    out_type = tuple(jax.ShapeDtypeStruct((NACC, CW), jnp.float32)
                     for _ in range(nchunks))

    def body(*refs):
        z_refs = refs[:nchunks]
        src_ref, dst_ref, zero_ref = refs[nchunks:nchunks + 3]
        out_refs = refs[nchunks + 3:2 * nchunks + 3]
        sidx, didx, rows, acc, semA, semB = refs[2 * nchunks + 3:]
        core = lax.axis_index("c")
        t = lax.axis_index("s")

        for c in range(nchunks):
            @pl.when(core == (c % 2))
            def _(c=c):
                pltpu.sync_copy(zero_ref, acc.at[pl.ds(t * STRIPE, STRIPE)])
                plsc.subcore_barrier()
                _edge_loop(z_refs[c], src_ref, dst_ref, t, sidx, didx, rows,
                           acc, (semA, semB), range(NWIN))
                plsc.subcore_barrier()
                pltpu.sync_copy(acc.at[pl.ds(t * STRIPE, STRIPE)],
                                out_refs[c].at[pl.ds(t * STRIPE, STRIPE)])
                plsc.subcore_barrier()

    return pl.kernel(body, out_type=out_type, mesh=_mesh(),
                     scratch_types=_segsum_scratch())


def _make_segsum_edges():
    out_type = (jax.ShapeDtypeStruct((NACC, CW), jnp.float32),
                jax.ShapeDtypeStruct((NACC, CW), jnp.float32))

    def body(z_ref, src_ref, dst_ref, zero_ref, out0, out1,
             sidx, didx, rows, acc, semA, semB):
        core = lax.axis_index("c")
        t = lax.axis_index("s")

        pltpu.sync_copy(zero_ref, acc.at[pl.ds(t * STRIPE, STRIPE)])
        plsc.subcore_barrier()
        for k, out_ref in enumerate((out0, out1)):
            @pl.when(core == k)
            def _(k=k, out_ref=out_ref):
                _edge_loop(z_ref, src_ref, dst_ref, t, sidx, didx, rows,
                           acc, (semA, semB), [k])
                plsc.subcore_barrier()
                pltpu.sync_copy(acc.at[pl.ds(t * STRIPE, STRIPE)],
                                out_ref.at[pl.ds(t * STRIPE, STRIPE)])

    return pl.kernel(body, out_type=out_type, mesh=_mesh(),
                     scratch_types=_segsum_scratch())


def _mm1_body(x_ref, w_ref, *out_refs):
    z = jnp.dot(x_ref[...], w_ref[...], preferred_element_type=jnp.float32)
    for c in range(NC1):
        out_refs[c][...] = z[:, c * CW:(c + 1) * CW]


def _mm2_body(*refs):
    a_refs = refs[:NC1]
    b_ref, w_ref = refs[NC1:NC1 + 2]
    out_refs = refs[NC1 + 2:]
    acc = None
    for c in range(NC1):
        xc = jnp.maximum(a_refs[c][...] + b_ref[0, c * CW:(c + 1) * CW], 0.0)
        p = jnp.dot(xc, w_ref[c * CW:(c + 1) * CW, :],
                    preferred_element_type=jnp.float32)
        acc = p if acc is None else acc + p
    for c in range(NC2):
        out_refs[c][...] = acc[:, c * CW:(c + 1) * CW]


def _mm3_body(a0, a1, a2, a3, b_ref, w3_ref, wi_ref, out_ref):
    a_refs = (a0, a1, a2, a3)
    acc = None
    for c in range(NC2):
        xc = jnp.maximum(a_refs[c][...] + b_ref[0, c * CW:(c + 1) * CW], 0.0)
        p = jnp.dot(xc, w3_ref[c * CW:(c + 1) * CW, :],
                    preferred_element_type=jnp.float32)
        acc = p if acc is None else acc + p
    out_ref[...] = jnp.dot(acc, wi_ref[...], preferred_element_type=jnp.float32)


def _final_body(p0, p1, b3_ref, wi_ref, bi_ref, out_ref):
    bv = jnp.dot(b3_ref[...], wi_ref[...],
                 preferred_element_type=jnp.float32) + bi_ref[...]
    h = p0[:, :64] + p1[:, :64] + bv
    out_ref[...] = jnp.maximum(h, 0.0)


def _row_spec(shape):
    return pl.BlockSpec((MM_BN,) + shape[1:], lambda i: (i,) + (0,) * (len(shape) - 1))


def _full_spec(shape):
    return pl.BlockSpec(shape, lambda i: (0,) * len(shape))


def kernel(features, edge_index, W1, b1, W2, b2, W3, b3, Wi, bi):
    f32 = jnp.float32
    W1p = jnp.pad(W1, ((0, 0), (0, NC1 * CW - 700)))
    b1p = jnp.pad(b1, (0, NC1 * CW - 700)).reshape(1, NC1 * CW)
    W2p = jnp.pad(W2, ((0, NC1 * CW - 700), (0, NC2 * CW - 400)))
    b2p = jnp.pad(b2, (0, NC2 * CW - 400)).reshape(1, NC2 * CW)
    W3p = jnp.pad(W3, ((0, NC2 * CW - 400), (0, 0)))
    Wip = jnp.pad(Wi, ((0, 0), (0, CW - 64)))
    b3r = b3.reshape(1, 100)
    bir = bi.reshape(1, 64)

    src = edge_index[0]
    dst = edge_index[1]
    pad = E_PAD - E
    srcp = jnp.concatenate([src, jnp.zeros((pad,), jnp.int32)]).reshape(NSUB, NBLK, BLK)
    dump = N + (jnp.arange(pad, dtype=jnp.int32) % (NACC - N))
    dstp = jnp.concatenate([dst, dump]).reshape(NSUB, NBLK, BLK)
    zeros = jnp.zeros((STRIPE, CW), f32)
    mm1 = pl.pallas_call(
        _mm1_body,
        grid=(MM_GRID,),
        in_specs=[_row_spec((N, 1433)), _full_spec((1433, NC1 * CW))],
        out_specs=[_row_spec((N, CW))] * NC1,
        out_shape=[jax.ShapeDtypeStruct((N, CW), f32)] * NC1,
    )
    z1 = mm1(features, W1p)

    a1 = _make_segsum_cols(NC1)(*z1, srcp, dstp, zeros)

    mm2 = pl.pallas_call(
        _mm2_body,
        grid=(MM_GRID,),
        in_specs=[_row_spec((NACC, CW))] * NC1 + [
            _full_spec((1, NC1 * CW)), _full_spec((NC1 * CW, NC2 * CW))],
        out_specs=[_row_spec((N, CW))] * NC2,
        out_shape=[jax.ShapeDtypeStruct((N, CW), f32)] * NC2,
    )
    z2 = mm2(*a1, b1p, W2p)

    a2 = _make_segsum_cols(NC2)(*z2, srcp, dstp, zeros)

    mm3 = pl.pallas_call(
        _mm3_body,
        grid=(MM_GRID,),
        in_specs=[_row_spec((NACC, CW))] * NC2 + [
            _full_spec((1, NC2 * CW)), _full_spec((NC2 * CW, 100)),
            _full_spec((100, CW))],
        out_specs=_row_spec((N, CW)),
        out_shape=jax.ShapeDtypeStruct((N, CW), f32),
    )
    z3 = mm3(*a2, b2p, W3p, Wip)

    a3 = _make_segsum_edges()(z3, srcp, dstp, zeros)

    fin = pl.pallas_call(
        _final_body,
        grid=(MM_GRID,),
        in_specs=[_row_spec((NACC, CW))] * 2 + [
            _full_spec((1, 100)), _full_spec((100, 64)), _full_spec((1, 64))],
        out_specs=_row_spec((N, 64)),
        out_shape=jax.ShapeDtypeStruct((N, 64), f32),
    )
    return fin(*a3, b3r, Wi, bir)
